# baseline (speedup 1.0000x reference)
import functools

import jax
import jax.numpy as jnp
from jax import lax
from jax.experimental import pallas as pl
from jax.experimental.pallas import tpu as pltpu
from jax.experimental.pallas import tpu_sc as plsc

N = 10000
E = 320000
D_NODE = 128
D_EDGE = 16
H = 64
D_OUT = 64

NC = 2
NS = 16
NW = NC * NS
EPW = E // NW
CHUNK = 80
NCHUNK = EPW // CHUNK
ROWS_PER_TILE = N // NS

_MESH = plsc.VectorSubcoreMesh(core_axis_name="c", subcore_axis_name="s")
_SC_PARAMS = pltpu.CompilerParams(use_tc_tiling_on_sc=False,
                                  needs_layout_passes=False)


def _worker_id():
    return lax.axis_index("s") * NC + lax.axis_index("c")


NB = 4


def _gather_body(xtab, btab, row3, col3, gx_out, gb_out,
                 idx_r, idx_c, gx_v, gb_v, sgx, sgb, swx, swb):
    wid = _worker_id()
    pltpu.sync_copy(row3.at[wid], idx_r)
    pltpu.sync_copy(col3.at[wid], idx_c)
    base = wid * EPW

    def issue(j, slot):
        pltpu.async_copy(xtab.at[idx_r.at[j]], gx_v.at[slot], sgx)
        pltpu.async_copy(btab.at[idx_c.at[j]], gb_v.at[slot], sgb)

    issue(0, 0)

    def chunk(j, carry):
        slot = lax.rem(j, NB)
        nslot = lax.rem(j + 1, NB)

        @pl.when(j + 1 < NCHUNK)
        def _prefetch():
            @pl.when(j + 1 >= NB)
            def _wait_wb():
                pltpu.make_async_copy(
                    gx_v.at[nslot], gx_out.at[pl.ds(base, CHUNK)], swx).wait()
                pltpu.make_async_copy(
                    gb_v.at[nslot], gb_out.at[pl.ds(base, CHUNK)], swb).wait()
            issue(j + 1, nslot)

        pltpu.make_async_copy(
            xtab.at[pl.ds(0, CHUNK)], gx_v.at[slot], sgx).wait()
        pltpu.make_async_copy(
            btab.at[pl.ds(0, CHUNK)], gb_v.at[slot], sgb).wait()
        off = base + j * CHUNK
        pltpu.async_copy(gx_v.at[slot], gx_out.at[pl.ds(off, CHUNK)], swx)
        pltpu.async_copy(gb_v.at[slot], gb_out.at[pl.ds(off, CHUNK)], swb)
        return carry

    lax.fori_loop(0, NCHUNK, chunk, 0)
    for _ in range(NB):
        pltpu.make_async_copy(
            gx_v.at[0], gx_out.at[pl.ds(base, CHUNK)], swx).wait()
        pltpu.make_async_copy(
            gb_v.at[0], gb_out.at[pl.ds(base, CHUNK)], swb).wait()


def _make_gather(dx):
    return pl.kernel(
        _gather_body,
        out_type=[
            jax.ShapeDtypeStruct((E, dx), jnp.bfloat16),
            jax.ShapeDtypeStruct((E, H), jnp.bfloat16),
        ],
        mesh=_MESH,
        scratch_types=[
            pltpu.VMEM((NCHUNK, CHUNK), jnp.int32),
            pltpu.VMEM((NCHUNK, CHUNK), jnp.int32),
            pltpu.VMEM((NB, CHUNK, dx), jnp.bfloat16),
            pltpu.VMEM((NB, CHUNK, H), jnp.bfloat16),
            pltpu.SemaphoreType.DMA,
            pltpu.SemaphoreType.DMA,
            pltpu.SemaphoreType.DMA,
            pltpu.SemaphoreType.DMA,
        ],
        compiler_params=_SC_PARAMS,
    )


def _scatter_body(m_hbm, col3, zeros2d, out, idx_c, m_v, acc, sld, sst):
    c = lax.axis_index("c")
    s = lax.axis_index("s")
    wid = s * NC + c
    tile_rows = pl.ds(s * ROWS_PER_TILE, ROWS_PER_TILE)
    pltpu.sync_copy(zeros2d.at[tile_rows], acc.at[tile_rows])
    pltpu.sync_copy(col3.at[wid], idx_c)
    plsc.subcore_barrier()
    base = wid * EPW

    def issue(j, slot):
        pltpu.async_copy(
            m_hbm.at[pl.ds(base + j * CHUNK, CHUNK)], m_v.at[slot], sld)

    issue(0, 0)

    def chunk(j, carry):
        slot = lax.rem(j, NB)
        nslot = lax.rem(j + 1, NB)

        @pl.when(j + 1 < NCHUNK)
        def _prefetch():
            @pl.when(j + 1 >= NB)
            def _wait_st():
                pltpu.make_async_copy(
                    m_v.at[nslot], acc.at[pl.ds(0, CHUNK)], sst).wait()
            issue(j + 1, nslot)

        pltpu.make_async_copy(
            m_hbm.at[pl.ds(base, CHUNK)], m_v.at[slot], sld).wait()
        pltpu.async_copy(m_v.at[slot], acc.at[idx_c.at[j]], sst, add=True)
        return carry

    lax.fori_loop(0, NCHUNK, chunk, 0)
    for _ in range(NB):
        pltpu.make_async_copy(
            m_v.at[0], acc.at[pl.ds(0, CHUNK)], sst).wait()
    plsc.subcore_barrier()
    pltpu.sync_copy(acc.at[tile_rows], out.at[c, tile_rows])


_scatter_call = pl.kernel(
    _scatter_body,
    out_type=[jax.ShapeDtypeStruct((NC, N, H), jnp.float32)],
    mesh=_MESH,
    scratch_types=[
        pltpu.VMEM((NCHUNK, CHUNK), jnp.int32),
        pltpu.VMEM((NB, CHUNK, H), jnp.float32),
        pltpu.VMEM_SHARED((N, H), jnp.float32),
        pltpu.SemaphoreType.DMA,
        pltpu.SemaphoreType.DMA,
    ],
    compiler_params=_SC_PARAMS,
)


def _counts_body(col2, zeros_n, out, col_v, cnt_v):
    wid = _worker_id()
    pltpu.sync_copy(zeros_n, cnt_v)
    pltpu.sync_copy(col2.at[wid], col_v)
    ones = jnp.ones((16,), jnp.float32)

    def step(i, carry):
        idx = col_v[pl.ds(i * 16, 16)]
        plsc.addupdate_scatter(cnt_v, [idx], ones)
        return carry

    lax.fori_loop(0, EPW // 16, step, 0)
    pltpu.sync_copy(cnt_v, out.at[wid])


_counts_call = pl.kernel(
    _counts_body,
    out_type=[jax.ShapeDtypeStruct((NW, N), jnp.float32)],
    mesh=_MESH,
    scratch_types=[
        pltpu.VMEM((EPW,), jnp.int32),
        pltpu.VMEM((N,), jnp.float32),
    ],
    compiler_params=_SC_PARAMS,
)


BE = 2000
BN = 2000


def _edge_tc_body(apply_relu, gx_ref, gb_ref, ein_ref, wsrc_ref, wee_ref,
                  w1x_ref, w1e_ref, be_ref, b1_ref, enew_ref, m_ref):
    gx = gx_ref[...]
    e = jnp.dot(gx, wsrc_ref[...], preferred_element_type=jnp.float32)
    e += gb_ref[...].astype(jnp.float32)
    e += jnp.dot(ein_ref[...], wee_ref[...], preferred_element_type=jnp.float32)
    e += be_ref[...]
    if apply_relu:
        e = jnp.maximum(e, 0.0)
    ebf = e.astype(jnp.bfloat16)
    enew_ref[...] = e.astype(enew_ref.dtype)
    m = jnp.dot(gx, w1x_ref[...], preferred_element_type=jnp.float32)
    m += jnp.dot(ebf, w1e_ref[...], preferred_element_type=jnp.float32)
    m += b1_ref[...]
    if apply_relu:
        m = jnp.maximum(m, 0.0)
    m_ref[...] = m


def _full(shape):
    return pl.BlockSpec(shape, lambda i: (0,) * len(shape))


def _make_edge_tc(dx, de, apply_relu, e_out_dtype):
    grid = (E // BE,)
    return pl.pallas_call(
        functools.partial(_edge_tc_body, apply_relu),
        grid=grid,
        in_specs=[
            pl.BlockSpec((BE, dx), lambda i: (i, 0)),
            pl.BlockSpec((BE, H), lambda i: (i, 0)),
            pl.BlockSpec((BE, de), lambda i: (i, 0)),
            _full((dx, H)),
            _full((de, H)),
            _full((dx, H)),
            _full((H, H)),
            _full((1, H)),
            _full((1, H)),
        ],
        out_specs=[
            pl.BlockSpec((BE, H), lambda i: (i, 0)),
            pl.BlockSpec((BE, H), lambda i: (i, 0)),
        ],
        out_shape=[
            jax.ShapeDtypeStruct((E, H), e_out_dtype),
            jax.ShapeDtypeStruct((E, H), jnp.float32),
        ],
    )


def _node_tc_body(apply_relu, has_next, sp_ref, cnt_ref, x_ref, w2x_ref,
                  w2a_ref, b2_ref, wdstn_ref, x_out, xbf_out, btab_out):
    sums = sp_ref[0] + sp_ref[1]
    cnt = jnp.sum(cnt_ref[...], axis=1)
    agg = sums / jnp.maximum(cnt, 1.0)[:, None]
    h = jnp.dot(x_ref[...], w2x_ref[...], preferred_element_type=jnp.float32)
    h += jnp.dot(agg, w2a_ref[...], preferred_element_type=jnp.float32)
    h += b2_ref[...]
    if apply_relu:
        h = jnp.maximum(h, 0.0)
    x_out[...] = h
    if has_next:
        xbf_out[...] = h.astype(jnp.bfloat16)
        btab_out[...] = jnp.dot(h, wdstn_ref[...],
                                preferred_element_type=jnp.float32
                                ).astype(jnp.bfloat16)
    else:
        xbf_out[...] = jnp.zeros_like(xbf_out)
        btab_out[...] = jnp.zeros_like(btab_out)


def _make_node_tc(dx, dout, apply_relu, has_next):
    grid = (N // BN,)
    return pl.pallas_call(
        functools.partial(_node_tc_body, apply_relu, has_next),
        grid=grid,
        in_specs=[
            pl.BlockSpec((NC, BN, H), lambda i: (0, i, 0)),
            pl.BlockSpec((BN, NW), lambda i: (i, 0)),
            pl.BlockSpec((BN, dx), lambda i: (i, 0)),
            _full((dx, dout)),
            _full((H, dout)),
            _full((1, dout)),
            _full((dout, H)),
        ],
        out_specs=[
            pl.BlockSpec((BN, dout), lambda i: (i, 0)),
            pl.BlockSpec((BN, dout), lambda i: (i, 0)),
            pl.BlockSpec((BN, H), lambda i: (i, 0)),
        ],
        out_shape=[
            jax.ShapeDtypeStruct((N, dout), jnp.float32),
            jax.ShapeDtypeStruct((N, dout), jnp.bfloat16),
            jax.ShapeDtypeStruct((N, H), jnp.bfloat16),
        ],
    )


def _btab_body(x_ref, wdst_ref, out_ref):
    out_ref[...] = jnp.dot(x_ref[...], wdst_ref[...],
                           preferred_element_type=jnp.float32
                           ).astype(jnp.bfloat16)


_btab_call = pl.pallas_call(
    _btab_body,
    grid=(N // BN,),
    in_specs=[
        pl.BlockSpec((BN, D_NODE), lambda i: (i, 0)),
        _full((D_NODE, H)),
    ],
    out_specs=pl.BlockSpec((BN, H), lambda i: (i, 0)),
    out_shape=jax.ShapeDtypeStruct((N, H), jnp.bfloat16),
)


_GATHER = {D_NODE: _make_gather(D_NODE), H: _make_gather(H)}
_EDGE_TC = [
    _make_edge_tc(D_NODE, D_EDGE, True, jnp.bfloat16),
    _make_edge_tc(H, H, True, jnp.bfloat16),
    _make_edge_tc(H, H, False, jnp.float32),
]
_NODE_TC = [
    _make_node_tc(D_NODE, H, True, True),
    _make_node_tc(H, H, True, True),
    _make_node_tc(H, D_OUT, False, False),
]


def kernel(x, edge_index, edge_attr, params):
    row = edge_index[0].reshape(NW, NCHUNK, CHUNK)
    col = edge_index[1]
    col3 = col.reshape(NW, NCHUNK, CHUNK)
    col2 = col.reshape(NW, EPW)
    zeros2d = jnp.zeros((N, H), jnp.float32)
    zeros_n = jnp.zeros((N,), jnp.float32)

    (cntp,) = _counts_call(col2, zeros_n)
    cntp = cntp.T

    dxs = [D_NODE, H, H]
    xl = x
    xbf = x.astype(jnp.bfloat16)
    e = edge_attr.astype(jnp.bfloat16)
    btab = _btab_call(x, params[0]["We"][dxs[0]:2 * dxs[0]])
    for i, p in enumerate(params):
        dx = dxs[i]
        we, w1, w2 = p["We"], p["W1"], p["W2"]
        wsrc = we[:dx].astype(jnp.bfloat16)
        wee = we[2 * dx:].astype(jnp.bfloat16)
        w1x = w1[:dx].astype(jnp.bfloat16)
        w1e = w1[dx:].astype(jnp.bfloat16)
        w2x = w2[:dx]
        w2a = w2[dx:]
        be = p["be"].reshape(1, -1)
        b1 = p["b1"].reshape(1, -1)
        b2 = p["b2"].reshape(1, -1)

        gx, gb = _GATHER[dx](xbf, btab, row, col3)
        enew, m = _EDGE_TC[i](gx, gb, e, wsrc, wee, w1x, w1e, be, b1)
        (sp,) = _scatter_call(m, col3, zeros2d)
        if i + 1 < len(params):
            wdstn = params[i + 1]["We"][dxs[i + 1]:2 * dxs[i + 1]]
        else:
            wdstn = jnp.zeros((D_OUT, H), jnp.float32)
        xl, xbf, btab = _NODE_TC[i](sp, cntp, xl, w2x, w2a, b2, wdstn)
        e = enew

    return xl, e


# speedup vs baseline: 1.1886x; 1.1886x over previous
import functools

import jax
import jax.numpy as jnp
from jax import lax
from jax.experimental import pallas as pl
from jax.experimental.pallas import tpu as pltpu
from jax.experimental.pallas import tpu_sc as plsc

N = 10000
E = 320000
D_NODE = 128
D_EDGE = 16
H = 64
D_OUT = 64

NC = 2
NS = 16
NW = NC * NS
EPW = E // NW
CHUNK = 80
NCHUNK = EPW // CHUNK
ROWS_PER_TILE = N // NS

_MESH = plsc.VectorSubcoreMesh(core_axis_name="c", subcore_axis_name="s")
_SC_PARAMS = pltpu.CompilerParams(use_tc_tiling_on_sc=False,
                                  needs_layout_passes=False)


def _worker_id():
    return lax.axis_index("s") * NC + lax.axis_index("c")


NB = 4


def _gather_body(xtab, btab, row3, col3, gx_out, gb_out,
                 idx_r, idx_c, gx_v, gb_v, sgx, sgb, swx, swb):
    wid = _worker_id()
    pltpu.sync_copy(row3.at[wid], idx_r)
    pltpu.sync_copy(col3.at[wid], idx_c)
    base = wid * EPW

    def issue(j, slot):
        pltpu.async_copy(xtab.at[idx_r.at[j]], gx_v.at[slot], sgx)
        pltpu.async_copy(btab.at[idx_c.at[j]], gb_v.at[slot], sgb)

    issue(0, 0)

    def chunk(j, carry):
        slot = lax.rem(j, NB)
        nslot = lax.rem(j + 1, NB)

        @pl.when(j + 1 < NCHUNK)
        def _prefetch():
            @pl.when(j + 1 >= NB)
            def _wait_wb():
                pltpu.make_async_copy(
                    gx_v.at[nslot], gx_out.at[pl.ds(base, CHUNK)], swx).wait()
                pltpu.make_async_copy(
                    gb_v.at[nslot], gb_out.at[pl.ds(base, CHUNK)], swb).wait()
            issue(j + 1, nslot)

        pltpu.make_async_copy(
            xtab.at[pl.ds(0, CHUNK)], gx_v.at[slot], sgx).wait()
        pltpu.make_async_copy(
            btab.at[pl.ds(0, CHUNK)], gb_v.at[slot], sgb).wait()
        off = base + j * CHUNK
        pltpu.async_copy(gx_v.at[slot], gx_out.at[pl.ds(off, CHUNK)], swx)
        pltpu.async_copy(gb_v.at[slot], gb_out.at[pl.ds(off, CHUNK)], swb)
        return carry

    lax.fori_loop(0, NCHUNK, chunk, 0)
    for _ in range(NB):
        pltpu.make_async_copy(
            gx_v.at[0], gx_out.at[pl.ds(base, CHUNK)], swx).wait()
        pltpu.make_async_copy(
            gb_v.at[0], gb_out.at[pl.ds(base, CHUNK)], swb).wait()


def _make_gather(dx):
    return pl.kernel(
        _gather_body,
        out_type=[
            jax.ShapeDtypeStruct((E, dx), jnp.float32),
            jax.ShapeDtypeStruct((E, H), jnp.float32),
        ],
        mesh=_MESH,
        scratch_types=[
            pltpu.VMEM((NCHUNK, CHUNK), jnp.int32),
            pltpu.VMEM((NCHUNK, CHUNK), jnp.int32),
            pltpu.VMEM((NB, CHUNK, dx), jnp.float32),
            pltpu.VMEM((NB, CHUNK, H), jnp.float32),
            pltpu.SemaphoreType.DMA,
            pltpu.SemaphoreType.DMA,
            pltpu.SemaphoreType.DMA,
            pltpu.SemaphoreType.DMA,
        ],
        compiler_params=_SC_PARAMS,
    )


def _scatter_body(m_hbm, col3, zeros2d, out, idx_c, m_v, acc, sld, sst):
    c = lax.axis_index("c")
    s = lax.axis_index("s")
    wid = s * NC + c
    tile_rows = pl.ds(s * ROWS_PER_TILE, ROWS_PER_TILE)
    pltpu.sync_copy(zeros2d.at[tile_rows], acc.at[tile_rows])
    pltpu.sync_copy(col3.at[wid], idx_c)
    plsc.subcore_barrier()
    base = wid * EPW

    def issue(j, slot):
        pltpu.async_copy(
            m_hbm.at[pl.ds(base + j * CHUNK, CHUNK)], m_v.at[slot], sld)

    issue(0, 0)

    def chunk(j, carry):
        slot = lax.rem(j, NB)
        nslot = lax.rem(j + 1, NB)

        @pl.when(j + 1 < NCHUNK)
        def _prefetch():
            @pl.when(j + 1 >= NB)
            def _wait_st():
                pltpu.make_async_copy(
                    m_v.at[nslot], acc.at[pl.ds(0, CHUNK)], sst).wait()
            issue(j + 1, nslot)

        pltpu.make_async_copy(
            m_hbm.at[pl.ds(base, CHUNK)], m_v.at[slot], sld).wait()
        pltpu.async_copy(m_v.at[slot], acc.at[idx_c.at[j]], sst, add=True)
        return carry

    lax.fori_loop(0, NCHUNK, chunk, 0)
    for _ in range(NB):
        pltpu.make_async_copy(
            m_v.at[0], acc.at[pl.ds(0, CHUNK)], sst).wait()
    plsc.subcore_barrier()
    pltpu.sync_copy(acc.at[tile_rows], out.at[c, tile_rows])


_scatter_call = pl.kernel(
    _scatter_body,
    out_type=[jax.ShapeDtypeStruct((NC, N, H), jnp.float32)],
    mesh=_MESH,
    scratch_types=[
        pltpu.VMEM((NCHUNK, CHUNK), jnp.int32),
        pltpu.VMEM((NB, CHUNK, H), jnp.float32),
        pltpu.VMEM_SHARED((N, H), jnp.float32),
        pltpu.SemaphoreType.DMA,
        pltpu.SemaphoreType.DMA,
    ],
    compiler_params=_SC_PARAMS,
)


def _counts_body(col2, zeros_n, out, col_v, cnt_v):
    wid = _worker_id()
    pltpu.sync_copy(zeros_n, cnt_v)
    pltpu.sync_copy(col2.at[wid], col_v)
    ones = jnp.ones((16,), jnp.float32)

    def step(i, carry):
        idx = col_v[pl.ds(i * 16, 16)]
        plsc.addupdate_scatter(cnt_v, [idx], ones)
        return carry

    lax.fori_loop(0, EPW // 16, step, 0)
    pltpu.sync_copy(cnt_v, out.at[wid])


_counts_call = pl.kernel(
    _counts_body,
    out_type=[jax.ShapeDtypeStruct((NW, N), jnp.float32)],
    mesh=_MESH,
    scratch_types=[
        pltpu.VMEM((EPW,), jnp.int32),
        pltpu.VMEM((N,), jnp.float32),
    ],
    compiler_params=_SC_PARAMS,
)


BE = 2000
BN = 2000


def _edge_tc_body(apply_relu, gx_ref, gb_ref, ein_ref, wsrc_ref, wee_ref,
                  w1x_ref, w1e_ref, be_ref, b1_ref, enew_ref, m_ref):
    gx = gx_ref[...]
    e = jnp.dot(gx, wsrc_ref[...], preferred_element_type=jnp.float32)
    e += gb_ref[...]
    e += jnp.dot(ein_ref[...], wee_ref[...], preferred_element_type=jnp.float32)
    e += be_ref[...]
    if apply_relu:
        e = jnp.maximum(e, 0.0)
    enew_ref[...] = e.astype(enew_ref.dtype)
    m = jnp.dot(gx, w1x_ref[...], preferred_element_type=jnp.float32)
    m += jnp.dot(e, w1e_ref[...], preferred_element_type=jnp.float32)
    m += b1_ref[...]
    if apply_relu:
        m = jnp.maximum(m, 0.0)
    m_ref[...] = m


def _full(shape):
    return pl.BlockSpec(shape, lambda i: (0,) * len(shape))


def _make_edge_tc(dx, de, apply_relu, e_out_dtype):
    grid = (E // BE,)
    return pl.pallas_call(
        functools.partial(_edge_tc_body, apply_relu),
        grid=grid,
        in_specs=[
            pl.BlockSpec((BE, dx), lambda i: (i, 0)),
            pl.BlockSpec((BE, H), lambda i: (i, 0)),
            pl.BlockSpec((BE, de), lambda i: (i, 0)),
            _full((dx, H)),
            _full((de, H)),
            _full((dx, H)),
            _full((H, H)),
            _full((1, H)),
            _full((1, H)),
        ],
        out_specs=[
            pl.BlockSpec((BE, H), lambda i: (i, 0)),
            pl.BlockSpec((BE, H), lambda i: (i, 0)),
        ],
        out_shape=[
            jax.ShapeDtypeStruct((E, H), jnp.float32),
            jax.ShapeDtypeStruct((E, H), jnp.float32),
        ],
    )


def _node_tc_body(apply_relu, has_next, sp_ref, cnt_ref, x_ref, w2x_ref,
                  w2a_ref, b2_ref, wdstn_ref, x_out, btab_out):
    sums = sp_ref[0] + sp_ref[1]
    cnt = jnp.sum(cnt_ref[...], axis=1)
    agg = sums / jnp.maximum(cnt, 1.0)[:, None]
    h = jnp.dot(x_ref[...], w2x_ref[...], preferred_element_type=jnp.float32)
    h += jnp.dot(agg, w2a_ref[...], preferred_element_type=jnp.float32)
    h += b2_ref[...]
    if apply_relu:
        h = jnp.maximum(h, 0.0)
    x_out[...] = h
    if has_next:
        btab_out[...] = jnp.dot(h, wdstn_ref[...],
                                preferred_element_type=jnp.float32)
    else:
        btab_out[...] = jnp.zeros_like(btab_out)


def _make_node_tc(dx, dout, apply_relu, has_next):
    grid = (N // BN,)
    return pl.pallas_call(
        functools.partial(_node_tc_body, apply_relu, has_next),
        grid=grid,
        in_specs=[
            pl.BlockSpec((NC, BN, H), lambda i: (0, i, 0)),
            pl.BlockSpec((BN, NW), lambda i: (i, 0)),
            pl.BlockSpec((BN, dx), lambda i: (i, 0)),
            _full((dx, dout)),
            _full((H, dout)),
            _full((1, dout)),
            _full((dout, H)),
        ],
        out_specs=[
            pl.BlockSpec((BN, dout), lambda i: (i, 0)),
            pl.BlockSpec((BN, H), lambda i: (i, 0)),
        ],
        out_shape=[
            jax.ShapeDtypeStruct((N, dout), jnp.float32),
            jax.ShapeDtypeStruct((N, H), jnp.float32),
        ],
    )


def _btab_body(x_ref, wdst_ref, out_ref):
    out_ref[...] = jnp.dot(x_ref[...], wdst_ref[...],
                           preferred_element_type=jnp.float32)


_btab_call = pl.pallas_call(
    _btab_body,
    grid=(N // BN,),
    in_specs=[
        pl.BlockSpec((BN, D_NODE), lambda i: (i, 0)),
        _full((D_NODE, H)),
    ],
    out_specs=pl.BlockSpec((BN, H), lambda i: (i, 0)),
    out_shape=jax.ShapeDtypeStruct((N, H), jnp.float32),
)


_GATHER = {D_NODE: _make_gather(D_NODE), H: _make_gather(H)}
_EDGE_TC = [
    _make_edge_tc(D_NODE, D_EDGE, True, jnp.float32),
    _make_edge_tc(H, H, True, jnp.float32),
    _make_edge_tc(H, H, False, jnp.float32),
]
_NODE_TC = [
    _make_node_tc(D_NODE, H, True, True),
    _make_node_tc(H, H, True, True),
    _make_node_tc(H, D_OUT, False, False),
]


def kernel(x, edge_index, edge_attr, params):
    row = edge_index[0].reshape(NW, NCHUNK, CHUNK)
    col = edge_index[1]
    col3 = col.reshape(NW, NCHUNK, CHUNK)
    col2 = col.reshape(NW, EPW)
    zeros2d = jnp.zeros((N, H), jnp.float32)
    zeros_n = jnp.zeros((N,), jnp.float32)

    (cntp,) = _counts_call(col2, zeros_n)
    cntp = cntp.T

    dxs = [D_NODE, H, H]
    xl = x
    xbf = x
    e = edge_attr
    btab = _btab_call(x, params[0]["We"][dxs[0]:2 * dxs[0]])
    for i, p in enumerate(params):
        dx = dxs[i]
        we, w1, w2 = p["We"], p["W1"], p["W2"]
        wsrc = we[:dx]
        wee = we[2 * dx:]
        w1x = w1[:dx]
        w1e = w1[dx:]
        w2x = w2[:dx]
        w2a = w2[dx:]
        be = p["be"].reshape(1, -1)
        b1 = p["b1"].reshape(1, -1)
        b2 = p["b2"].reshape(1, -1)

        gx, gb = _GATHER[dx](xbf, btab, row, col3)
        enew, m = _EDGE_TC[i](gx, gb, e, wsrc, wee, w1x, w1e, be, b1)
        (sp,) = _scatter_call(m, col3, zeros2d)
        if i + 1 < len(params):
            wdstn = params[i + 1]["We"][dxs[i + 1]:2 * dxs[i + 1]]
        else:
            wdstn = jnp.zeros((D_OUT, H), jnp.float32)
        xl, btab = _NODE_TC[i](sp, cntp, xl, w2x, w2a, b2, wdstn)
        xbf = xl
        e = enew

    return xl, e


# speedup vs baseline: 1.8551x; 1.5608x over previous
import functools

import jax
import jax.numpy as jnp
from jax import lax
from jax.experimental import pallas as pl
from jax.experimental.pallas import tpu as pltpu
from jax.experimental.pallas import tpu_sc as plsc

N = 10000
E = 320000
D_NODE = 128
D_EDGE = 16
H = 64
D_OUT = 64

NC = 2
NS = 16
NW = NC * NS
EPW = E // NW
CHUNK = 80
NCHUNK = EPW // CHUNK
ROWS_PER_TILE = N // NS

_MESH = plsc.VectorSubcoreMesh(core_axis_name="c", subcore_axis_name="s")
_SC_PARAMS = pltpu.CompilerParams(use_tc_tiling_on_sc=False,
                                  needs_layout_passes=False)


def _worker_id():
    return lax.axis_index("s") * NC + lax.axis_index("c")


NB = 4


def _gather_body(xtab, btab, row3, col3, gx_out, gb_out,
                 idx_r, idx_c, gx_v, gb_v, sgx, sgb, swx, swb):
    wid = _worker_id()
    pltpu.sync_copy(row3.at[wid], idx_r)
    pltpu.sync_copy(col3.at[wid], idx_c)
    base = wid * EPW

    def issue(j, slot):
        pltpu.async_copy(xtab.at[idx_r.at[j]], gx_v.at[slot], sgx)
        pltpu.async_copy(btab.at[idx_c.at[j]], gb_v.at[slot], sgb)

    issue(0, 0)

    def chunk(j, carry):
        slot = lax.rem(j, NB)
        nslot = lax.rem(j + 1, NB)

        @pl.when(j + 1 < NCHUNK)
        def _prefetch():
            @pl.when(j + 1 >= NB)
            def _wait_wb():
                pltpu.make_async_copy(
                    gx_v.at[nslot], gx_out.at[pl.ds(base, CHUNK)], swx).wait()
                pltpu.make_async_copy(
                    gb_v.at[nslot], gb_out.at[pl.ds(base, CHUNK)], swb).wait()
            issue(j + 1, nslot)

        pltpu.make_async_copy(
            xtab.at[pl.ds(0, CHUNK)], gx_v.at[slot], sgx).wait()
        pltpu.make_async_copy(
            btab.at[pl.ds(0, CHUNK)], gb_v.at[slot], sgb).wait()
        off = base + j * CHUNK
        pltpu.async_copy(gx_v.at[slot], gx_out.at[pl.ds(off, CHUNK)], swx)
        pltpu.async_copy(gb_v.at[slot], gb_out.at[pl.ds(off, CHUNK)], swb)
        return carry

    lax.fori_loop(0, NCHUNK, chunk, 0)
    for _ in range(NB):
        pltpu.make_async_copy(
            gx_v.at[0], gx_out.at[pl.ds(base, CHUNK)], swx).wait()
        pltpu.make_async_copy(
            gb_v.at[0], gb_out.at[pl.ds(base, CHUNK)], swb).wait()


def _make_gather(dx):
    return pl.kernel(
        _gather_body,
        out_type=[
            jax.ShapeDtypeStruct((E, dx), jnp.float32),
            jax.ShapeDtypeStruct((E, H), jnp.float32),
        ],
        mesh=_MESH,
        scratch_types=[
            pltpu.VMEM((NCHUNK, CHUNK), jnp.int32),
            pltpu.VMEM((NCHUNK, CHUNK), jnp.int32),
            pltpu.VMEM((NB, CHUNK, dx), jnp.float32),
            pltpu.VMEM((NB, CHUNK, H), jnp.float32),
            pltpu.SemaphoreType.DMA,
            pltpu.SemaphoreType.DMA,
            pltpu.SemaphoreType.DMA,
            pltpu.SemaphoreType.DMA,
        ],
        compiler_params=_SC_PARAMS,
    )


def _scatter_body(m_hbm, col3, zeros2d, out, idx_c, m_v, acc, sld, sst):
    c = lax.axis_index("c")
    s = lax.axis_index("s")
    wid = s * NC + c
    tile_rows = pl.ds(s * ROWS_PER_TILE, ROWS_PER_TILE)
    pltpu.sync_copy(zeros2d.at[tile_rows], acc.at[tile_rows])
    pltpu.sync_copy(col3.at[wid], idx_c)
    plsc.subcore_barrier()
    base = wid * EPW

    def issue(j, slot):
        pltpu.async_copy(
            m_hbm.at[pl.ds(base + j * CHUNK, CHUNK)], m_v.at[slot], sld)

    issue(0, 0)

    def chunk(j, carry):
        slot = lax.rem(j, NB)
        nslot = lax.rem(j + 1, NB)

        @pl.when(j + 1 < NCHUNK)
        def _prefetch():
            @pl.when(j + 1 >= NB)
            def _wait_st():
                pltpu.make_async_copy(
                    m_v.at[nslot], acc.at[pl.ds(0, CHUNK)], sst).wait()
            issue(j + 1, nslot)

        pltpu.make_async_copy(
            m_hbm.at[pl.ds(base, CHUNK)], m_v.at[slot], sld).wait()
        pltpu.async_copy(m_v.at[slot], acc.at[idx_c.at[j]], sst, add=True)
        return carry

    lax.fori_loop(0, NCHUNK, chunk, 0)
    for _ in range(NB):
        pltpu.make_async_copy(
            m_v.at[0], acc.at[pl.ds(0, CHUNK)], sst).wait()
    plsc.subcore_barrier()
    pltpu.sync_copy(acc.at[tile_rows], out.at[c, tile_rows])


_scatter_call = pl.kernel(
    _scatter_body,
    out_type=[jax.ShapeDtypeStruct((NC, N, H), jnp.float32)],
    mesh=_MESH,
    scratch_types=[
        pltpu.VMEM((NCHUNK, CHUNK), jnp.int32),
        pltpu.VMEM((NB, CHUNK, H), jnp.float32),
        pltpu.VMEM_SHARED((N, H), jnp.float32),
        pltpu.SemaphoreType.DMA,
        pltpu.SemaphoreType.DMA,
    ],
    compiler_params=_SC_PARAMS,
)


def _counts_body(col2, zeros_n, out, col_v, cnt_v):
    wid = _worker_id()
    pltpu.sync_copy(zeros_n, cnt_v)
    pltpu.sync_copy(col2.at[wid], col_v)
    ones = jnp.ones((16,), jnp.float32)

    def step(i, carry):
        idx = col_v[pl.ds(i * 16, 16)]
        plsc.addupdate_scatter(cnt_v, [idx], ones)
        return carry

    lax.fori_loop(0, EPW // 16, step, 0)
    pltpu.sync_copy(cnt_v, out.at[wid])


_counts_call = pl.kernel(
    _counts_body,
    out_type=[jax.ShapeDtypeStruct((NW, N), jnp.float32)],
    mesh=_MESH,
    scratch_types=[
        pltpu.VMEM((EPW,), jnp.int32),
        pltpu.VMEM((N,), jnp.float32),
    ],
    compiler_params=_SC_PARAMS,
)


BEP = 1000
EP = E // 2
BN = 2000


def _edge_tc_body(apply_relu, gx_ref, gb_ref, ein_ref, wsrc_ref, wee_ref,
                  w1x_ref, w1e_ref, be_ref, b1_ref, enew_ref, m_ref):
    gx = gx_ref[...]
    e = jnp.dot(gx, wsrc_ref[...], preferred_element_type=jnp.float32)
    e += gb_ref[...]
    e += jnp.dot(ein_ref[...], wee_ref[...], preferred_element_type=jnp.float32)
    e += be_ref[...]
    if apply_relu:
        e = jnp.maximum(e, 0.0)
    enew_ref[...] = e.astype(enew_ref.dtype)
    m = jnp.dot(gx, w1x_ref[...], preferred_element_type=jnp.float32)
    m += jnp.dot(e, w1e_ref[...], preferred_element_type=jnp.float32)
    m += b1_ref[...]
    if apply_relu:
        m = jnp.maximum(m, 0.0)
    m_ref[...] = m


def _full(shape):
    return pl.BlockSpec(shape, lambda i: (0,) * len(shape))


def _make_edge_tc(dx, de, apply_relu):
    grid = (EP // BEP,)
    return pl.pallas_call(
        functools.partial(_edge_tc_body, apply_relu),
        grid=grid,
        in_specs=[
            pl.BlockSpec((BEP, 2 * dx), lambda i: (i, 0)),
            pl.BlockSpec((BEP, 2 * H), lambda i: (i, 0)),
            pl.BlockSpec((BEP, 2 * de), lambda i: (i, 0)),
            _full((2 * dx, 2 * H)),
            _full((2 * de, 2 * H)),
            _full((2 * dx, 2 * H)),
            _full((2 * H, 2 * H)),
            _full((1, 2 * H)),
            _full((1, 2 * H)),
        ],
        out_specs=[
            pl.BlockSpec((BEP, 2 * H), lambda i: (i, 0)),
            pl.BlockSpec((BEP, 2 * H), lambda i: (i, 0)),
        ],
        out_shape=[
            jax.ShapeDtypeStruct((EP, 2 * H), jnp.float32),
            jax.ShapeDtypeStruct((EP, 2 * H), jnp.float32),
        ],
    )


def _node_tc_body(apply_relu, has_next, sp_ref, cnt_ref, x_ref, w2x_ref,
                  w2a_ref, b2_ref, wdstn_ref, x_out, btab_out):
    sums = sp_ref[0] + sp_ref[1]
    cnt = jnp.sum(cnt_ref[...], axis=1)
    agg = sums / jnp.maximum(cnt, 1.0)[:, None]
    h = jnp.dot(x_ref[...], w2x_ref[...], preferred_element_type=jnp.float32)
    h += jnp.dot(agg, w2a_ref[...], preferred_element_type=jnp.float32)
    h += b2_ref[...]
    if apply_relu:
        h = jnp.maximum(h, 0.0)
    x_out[...] = h
    if has_next:
        btab_out[...] = jnp.dot(h, wdstn_ref[...],
                                preferred_element_type=jnp.float32)
    else:
        btab_out[...] = jnp.zeros_like(btab_out)


def _make_node_tc(dx, dout, apply_relu, has_next):
    grid = (N // BN,)
    return pl.pallas_call(
        functools.partial(_node_tc_body, apply_relu, has_next),
        grid=grid,
        in_specs=[
            pl.BlockSpec((NC, BN, H), lambda i: (0, i, 0)),
            pl.BlockSpec((BN, NW), lambda i: (i, 0)),
            pl.BlockSpec((BN, dx), lambda i: (i, 0)),
            _full((dx, dout)),
            _full((H, dout)),
            _full((1, dout)),
            _full((dout, H)),
        ],
        out_specs=[
            pl.BlockSpec((BN, dout), lambda i: (i, 0)),
            pl.BlockSpec((BN, H), lambda i: (i, 0)),
        ],
        out_shape=[
            jax.ShapeDtypeStruct((N, dout), jnp.float32),
            jax.ShapeDtypeStruct((N, H), jnp.float32),
        ],
    )


def _btab_body(x_ref, wdst_ref, out_ref):
    out_ref[...] = jnp.dot(x_ref[...], wdst_ref[...],
                           preferred_element_type=jnp.float32)


_btab_call = pl.pallas_call(
    _btab_body,
    grid=(N // BN,),
    in_specs=[
        pl.BlockSpec((BN, D_NODE), lambda i: (i, 0)),
        _full((D_NODE, H)),
    ],
    out_specs=pl.BlockSpec((BN, H), lambda i: (i, 0)),
    out_shape=jax.ShapeDtypeStruct((N, H), jnp.float32),
)


_GATHER = {D_NODE: _make_gather(D_NODE), H: _make_gather(H)}
_EDGE_TC = [
    _make_edge_tc(D_NODE, D_EDGE, True),
    _make_edge_tc(H, H, True),
    _make_edge_tc(H, H, False),
]


def _blockdiag(w):
    z = jnp.zeros_like(w)
    return jnp.concatenate(
        [jnp.concatenate([w, z], axis=1), jnp.concatenate([z, w], axis=1)],
        axis=0)
_NODE_TC = [
    _make_node_tc(D_NODE, H, True, True),
    _make_node_tc(H, H, True, True),
    _make_node_tc(H, D_OUT, False, False),
]


def kernel(x, edge_index, edge_attr, params):
    row = edge_index[0].reshape(NW, NCHUNK, CHUNK)
    col = edge_index[1]
    col3 = col.reshape(NW, NCHUNK, CHUNK)
    col2 = col.reshape(NW, EPW)
    zeros2d = jnp.zeros((N, H), jnp.float32)
    zeros_n = jnp.zeros((N,), jnp.float32)

    (cntp,) = _counts_call(col2, zeros_n)
    cntp = cntp.T

    dxs = [D_NODE, H, H]
    xl = x
    e2 = edge_attr.reshape(EP, 2 * D_EDGE)
    btab = _btab_call(x, params[0]["We"][dxs[0]:2 * dxs[0]])
    for i, p in enumerate(params):
        dx = dxs[i]
        we, w1, w2 = p["We"], p["W1"], p["W2"]
        wsrc = _blockdiag(we[:dx])
        wee = _blockdiag(we[2 * dx:])
        w1x = _blockdiag(w1[:dx])
        w1e = _blockdiag(w1[dx:])
        w2x = w2[:dx]
        w2a = w2[dx:]
        be = jnp.tile(p["be"].reshape(1, -1), (1, 2))
        b1 = jnp.tile(p["b1"].reshape(1, -1), (1, 2))
        b2 = p["b2"].reshape(1, -1)

        gx, gb = _GATHER[dx](xl, btab, row, col3)
        gx2 = gx.reshape(EP, 2 * dx)
        gb2 = gb.reshape(EP, 2 * H)
        enew2, m2 = _EDGE_TC[i](gx2, gb2, e2, wsrc, wee, w1x, w1e, be, b1)
        (sp,) = _scatter_call(m2.reshape(E, H), col3, zeros2d)
        if i + 1 < len(params):
            wdstn = params[i + 1]["We"][dxs[i + 1]:2 * dxs[i + 1]]
        else:
            wdstn = jnp.zeros((D_OUT, H), jnp.float32)
        xl, btab = _NODE_TC[i](sp, cntp, xl, w2x, w2a, b2, wdstn)
        e2 = enew2

    return xl, e2.reshape(E, H)


# speedup vs baseline: 2.0051x; 1.0809x over previous
import functools

import jax
import jax.numpy as jnp
from jax import lax
from jax.experimental import pallas as pl
from jax.experimental.pallas import tpu as pltpu
from jax.experimental.pallas import tpu_sc as plsc

N = 10000
E = 320000
D_NODE = 128
D_EDGE = 16
H = 64
D_OUT = 64

NC = 2
NS = 16
NW = NC * NS
EPW = E // NW
CHUNK = 80
NCHUNK = EPW // CHUNK
ROWS_PER_TILE = N // NS

_MESH = plsc.VectorSubcoreMesh(core_axis_name="c", subcore_axis_name="s")
_SC_PARAMS = pltpu.CompilerParams(use_tc_tiling_on_sc=False,
                                  needs_layout_passes=False)


def _worker_id():
    return lax.axis_index("s") * NC + lax.axis_index("c")


NB = 4


def _gather_body(xtab, btab, row3, col3, gx_out, gb_out,
                 idx_r, idx_c, gx_v, gb_v, sgx, sgb, swx, swb):
    wid = _worker_id()
    pltpu.sync_copy(row3.at[wid], idx_r)
    pltpu.sync_copy(col3.at[wid], idx_c)
    base = wid * EPW

    def issue(j, slot):
        pltpu.async_copy(xtab.at[idx_r.at[j]], gx_v.at[slot], sgx)
        pltpu.async_copy(btab.at[idx_c.at[j]], gb_v.at[slot], sgb)

    issue(0, 0)

    def chunk(j, carry):
        slot = lax.rem(j, NB)
        nslot = lax.rem(j + 1, NB)

        @pl.when(j + 1 < NCHUNK)
        def _prefetch():
            @pl.when(j + 1 >= NB)
            def _wait_wb():
                pltpu.make_async_copy(
                    gx_v.at[nslot], gx_out.at[pl.ds(base, CHUNK)], swx).wait()
                pltpu.make_async_copy(
                    gb_v.at[nslot], gb_out.at[pl.ds(base, CHUNK)], swb).wait()
            issue(j + 1, nslot)

        pltpu.make_async_copy(
            xtab.at[pl.ds(0, CHUNK)], gx_v.at[slot], sgx).wait()
        pltpu.make_async_copy(
            btab.at[pl.ds(0, CHUNK)], gb_v.at[slot], sgb).wait()
        off = base + j * CHUNK
        pltpu.async_copy(gx_v.at[slot], gx_out.at[pl.ds(off, CHUNK)], swx)
        pltpu.async_copy(gb_v.at[slot], gb_out.at[pl.ds(off, CHUNK)], swb)
        return carry

    lax.fori_loop(0, NCHUNK, chunk, 0)
    for _ in range(NB):
        pltpu.make_async_copy(
            gx_v.at[0], gx_out.at[pl.ds(base, CHUNK)], swx).wait()
        pltpu.make_async_copy(
            gb_v.at[0], gb_out.at[pl.ds(base, CHUNK)], swb).wait()


def _make_gather(dx):
    return pl.kernel(
        _gather_body,
        out_type=[
            jax.ShapeDtypeStruct((E, dx), jnp.float32),
            jax.ShapeDtypeStruct((E, H), jnp.float32),
        ],
        mesh=_MESH,
        scratch_types=[
            pltpu.VMEM((NCHUNK, CHUNK), jnp.int32),
            pltpu.VMEM((NCHUNK, CHUNK), jnp.int32),
            pltpu.VMEM((NB, CHUNK, dx), jnp.float32),
            pltpu.VMEM((NB, CHUNK, H), jnp.float32),
            pltpu.SemaphoreType.DMA,
            pltpu.SemaphoreType.DMA,
            pltpu.SemaphoreType.DMA,
            pltpu.SemaphoreType.DMA,
        ],
        compiler_params=_SC_PARAMS,
    )


def _gather3_body(taba, tabb, btab, row3, col3, oa, ob, oc,
                  idx_r, idx_c, va, vb, vc, sa, sb, sc, swa, swb, swc):
    wid = _worker_id()
    pltpu.sync_copy(row3.at[wid], idx_r)
    pltpu.sync_copy(col3.at[wid], idx_c)
    base = wid * EPW
    tabs = (taba, tabb, btab)
    bufs = (va, vb, vc)
    gsems = (sa, sb, sc)
    wsems = (swa, swb, swc)
    outs = (oa, ob, oc)

    def issue(j, slot):
        pltpu.async_copy(taba.at[idx_r.at[j]], va.at[slot], sa)
        pltpu.async_copy(tabb.at[idx_r.at[j]], vb.at[slot], sb)
        pltpu.async_copy(btab.at[idx_c.at[j]], vc.at[slot], sc)

    issue(0, 0)

    def chunk(j, carry):
        slot = lax.rem(j, NB)
        nslot = lax.rem(j + 1, NB)

        @pl.when(j + 1 < NCHUNK)
        def _prefetch():
            @pl.when(j + 1 >= NB)
            def _wait_wb():
                for v, o, sw in zip(bufs, outs, wsems):
                    pltpu.make_async_copy(
                        v.at[nslot], o.at[pl.ds(base, CHUNK)], sw).wait()
            issue(j + 1, nslot)

        for t, v, sg in zip(tabs, bufs, gsems):
            pltpu.make_async_copy(t.at[pl.ds(0, CHUNK)], v.at[slot], sg).wait()
        off = base + j * CHUNK
        for v, o, sw in zip(bufs, outs, wsems):
            pltpu.async_copy(v.at[slot], o.at[pl.ds(off, CHUNK)], sw)
        return carry

    lax.fori_loop(0, NCHUNK, chunk, 0)
    for _ in range(NB):
        for v, o, sw in zip(bufs, outs, wsems):
            pltpu.make_async_copy(
                v.at[0], o.at[pl.ds(base, CHUNK)], sw).wait()


_gather3_call = pl.kernel(
    _gather3_body,
    out_type=[jax.ShapeDtypeStruct((E, H), jnp.float32)] * 3,
    mesh=_MESH,
    scratch_types=(
        [pltpu.VMEM((NCHUNK, CHUNK), jnp.int32)] * 2
        + [pltpu.VMEM((NB, CHUNK, H), jnp.float32)] * 3
        + [pltpu.SemaphoreType.DMA] * 6
    ),
    compiler_params=_SC_PARAMS,
)


def _scatter_body(m_hbm, col3, zeros2d, out, idx_c, m_v, acc, sld, sst):
    c = lax.axis_index("c")
    s = lax.axis_index("s")
    wid = s * NC + c
    tile_rows = pl.ds(s * ROWS_PER_TILE, ROWS_PER_TILE)
    pltpu.sync_copy(zeros2d.at[tile_rows], acc.at[tile_rows])
    pltpu.sync_copy(col3.at[wid], idx_c)
    plsc.subcore_barrier()
    base = wid * EPW

    def issue(j, slot):
        pltpu.async_copy(
            m_hbm.at[pl.ds(base + j * CHUNK, CHUNK)], m_v.at[slot], sld)

    issue(0, 0)

    def chunk(j, carry):
        slot = lax.rem(j, NB)
        nslot = lax.rem(j + 1, NB)

        @pl.when(j + 1 < NCHUNK)
        def _prefetch():
            @pl.when(j + 1 >= NB)
            def _wait_st():
                pltpu.make_async_copy(
                    m_v.at[nslot], acc.at[pl.ds(0, CHUNK)], sst).wait()
            issue(j + 1, nslot)

        pltpu.make_async_copy(
            m_hbm.at[pl.ds(base, CHUNK)], m_v.at[slot], sld).wait()
        pltpu.async_copy(m_v.at[slot], acc.at[idx_c.at[j]], sst, add=True)
        return carry

    lax.fori_loop(0, NCHUNK, chunk, 0)
    for _ in range(NB):
        pltpu.make_async_copy(
            m_v.at[0], acc.at[pl.ds(0, CHUNK)], sst).wait()
    plsc.subcore_barrier()
    pltpu.sync_copy(acc.at[tile_rows], out.at[c, tile_rows])


_scatter_call = pl.kernel(
    _scatter_body,
    out_type=[jax.ShapeDtypeStruct((NC, N, H), jnp.float32)],
    mesh=_MESH,
    scratch_types=[
        pltpu.VMEM((NCHUNK, CHUNK), jnp.int32),
        pltpu.VMEM((NB, CHUNK, H), jnp.float32),
        pltpu.VMEM_SHARED((N, H), jnp.float32),
        pltpu.SemaphoreType.DMA,
        pltpu.SemaphoreType.DMA,
    ],
    compiler_params=_SC_PARAMS,
)


def _counts_body(col2, zeros_n, out, col_v, cnt_v):
    wid = _worker_id()
    pltpu.sync_copy(zeros_n, cnt_v)
    pltpu.sync_copy(col2.at[wid], col_v)
    ones = jnp.ones((16,), jnp.float32)

    def step(i, carry):
        idx = col_v[pl.ds(i * 16, 16)]
        plsc.addupdate_scatter(cnt_v, [idx], ones)
        return carry

    lax.fori_loop(0, EPW // 16, step, 0)
    pltpu.sync_copy(cnt_v, out.at[wid])


_counts_call = pl.kernel(
    _counts_body,
    out_type=[jax.ShapeDtypeStruct((NW, N), jnp.float32)],
    mesh=_MESH,
    scratch_types=[
        pltpu.VMEM((EPW,), jnp.int32),
        pltpu.VMEM((N,), jnp.float32),
    ],
    compiler_params=_SC_PARAMS,
)


BEP = 1000
EP = E // 2
BN = 2000


def _edge_tc_body(apply_relu, gx_ref, gb_ref, ein_ref, wsrc_ref, wee_ref,
                  w1x_ref, w1e_ref, be_ref, b1_ref, enew_ref, m_ref):
    gx = gx_ref[...]
    e = jnp.dot(gx, wsrc_ref[...], preferred_element_type=jnp.float32)
    e += gb_ref[...]
    e += jnp.dot(ein_ref[...], wee_ref[...], preferred_element_type=jnp.float32)
    e += be_ref[...]
    if apply_relu:
        e = jnp.maximum(e, 0.0)
    enew_ref[...] = e.astype(enew_ref.dtype)
    m = jnp.dot(gx, w1x_ref[...], preferred_element_type=jnp.float32)
    m += jnp.dot(e, w1e_ref[...], preferred_element_type=jnp.float32)
    m += b1_ref[...]
    if apply_relu:
        m = jnp.maximum(m, 0.0)
    m_ref[...] = m


def _full(shape):
    return pl.BlockSpec(shape, lambda i: (0,) * len(shape))


def _edge1_tc_body(gxa_ref, gxb_ref, gb_ref, ein_ref, wsa_ref, wsb_ref,
                   wee_ref, w1a_ref, w1b_ref, w1e_ref, be_ref, b1_ref,
                   enew_ref, m_ref):
    gxa = gxa_ref[...]
    gxb = gxb_ref[...]
    e = jnp.dot(gxa, wsa_ref[...], preferred_element_type=jnp.float32)
    e += jnp.dot(gxb, wsb_ref[...], preferred_element_type=jnp.float32)
    e += gb_ref[...]
    e += jnp.dot(ein_ref[...], wee_ref[...], preferred_element_type=jnp.float32)
    e += be_ref[...]
    e = jnp.maximum(e, 0.0)
    enew_ref[...] = e
    m = jnp.dot(gxa, w1a_ref[...], preferred_element_type=jnp.float32)
    m += jnp.dot(gxb, w1b_ref[...], preferred_element_type=jnp.float32)
    m += jnp.dot(e, w1e_ref[...], preferred_element_type=jnp.float32)
    m += b1_ref[...]
    m = jnp.maximum(m, 0.0)
    m_ref[...] = m


_edge1_tc_call = pl.pallas_call(
    _edge1_tc_body,
    grid=(EP // BEP,),
    in_specs=[
        pl.BlockSpec((BEP, 2 * H), lambda i: (i, 0)),
        pl.BlockSpec((BEP, 2 * H), lambda i: (i, 0)),
        pl.BlockSpec((BEP, 2 * H), lambda i: (i, 0)),
        pl.BlockSpec((BEP, 2 * D_EDGE), lambda i: (i, 0)),
        _full((2 * H, 2 * H)),
        _full((2 * H, 2 * H)),
        _full((2 * D_EDGE, 2 * H)),
        _full((2 * H, 2 * H)),
        _full((2 * H, 2 * H)),
        _full((2 * H, 2 * H)),
        _full((1, 2 * H)),
        _full((1, 2 * H)),
    ],
    out_specs=[
        pl.BlockSpec((BEP, 2 * H), lambda i: (i, 0)),
        pl.BlockSpec((BEP, 2 * H), lambda i: (i, 0)),
    ],
    out_shape=[
        jax.ShapeDtypeStruct((EP, 2 * H), jnp.float32),
        jax.ShapeDtypeStruct((EP, 2 * H), jnp.float32),
    ],
)


def _make_edge_tc(dx, de, apply_relu):
    grid = (EP // BEP,)
    return pl.pallas_call(
        functools.partial(_edge_tc_body, apply_relu),
        grid=grid,
        in_specs=[
            pl.BlockSpec((BEP, 2 * dx), lambda i: (i, 0)),
            pl.BlockSpec((BEP, 2 * H), lambda i: (i, 0)),
            pl.BlockSpec((BEP, 2 * de), lambda i: (i, 0)),
            _full((2 * dx, 2 * H)),
            _full((2 * de, 2 * H)),
            _full((2 * dx, 2 * H)),
            _full((2 * H, 2 * H)),
            _full((1, 2 * H)),
            _full((1, 2 * H)),
        ],
        out_specs=[
            pl.BlockSpec((BEP, 2 * H), lambda i: (i, 0)),
            pl.BlockSpec((BEP, 2 * H), lambda i: (i, 0)),
        ],
        out_shape=[
            jax.ShapeDtypeStruct((EP, 2 * H), jnp.float32),
            jax.ShapeDtypeStruct((EP, 2 * H), jnp.float32),
        ],
    )


def _node_tc_body(apply_relu, has_next, sp_ref, cnt_ref, x_ref, w2x_ref,
                  w2a_ref, b2_ref, wdstn_ref, x_out, btab_out):
    sums = sp_ref[0] + sp_ref[1]
    cnt = jnp.sum(cnt_ref[...], axis=1)
    agg = sums / jnp.maximum(cnt, 1.0)[:, None]
    h = jnp.dot(x_ref[...], w2x_ref[...], preferred_element_type=jnp.float32)
    h += jnp.dot(agg, w2a_ref[...], preferred_element_type=jnp.float32)
    h += b2_ref[...]
    if apply_relu:
        h = jnp.maximum(h, 0.0)
    x_out[...] = h
    if has_next:
        btab_out[...] = jnp.dot(h, wdstn_ref[...],
                                preferred_element_type=jnp.float32)
    else:
        btab_out[...] = jnp.zeros_like(btab_out)


def _make_node_tc(dx, dout, apply_relu, has_next):
    grid = (N // BN,)
    return pl.pallas_call(
        functools.partial(_node_tc_body, apply_relu, has_next),
        grid=grid,
        in_specs=[
            pl.BlockSpec((NC, BN, H), lambda i: (0, i, 0)),
            pl.BlockSpec((BN, NW), lambda i: (i, 0)),
            pl.BlockSpec((BN, dx), lambda i: (i, 0)),
            _full((dx, dout)),
            _full((H, dout)),
            _full((1, dout)),
            _full((dout, H)),
        ],
        out_specs=[
            pl.BlockSpec((BN, dout), lambda i: (i, 0)),
            pl.BlockSpec((BN, H), lambda i: (i, 0)),
        ],
        out_shape=[
            jax.ShapeDtypeStruct((N, dout), jnp.float32),
            jax.ShapeDtypeStruct((N, H), jnp.float32),
        ],
    )


def _btab_body(x_ref, wdst_ref, out_ref):
    out_ref[...] = jnp.dot(x_ref[...], wdst_ref[...],
                           preferred_element_type=jnp.float32)


_btab_call = pl.pallas_call(
    _btab_body,
    grid=(N // BN,),
    in_specs=[
        pl.BlockSpec((BN, D_NODE), lambda i: (i, 0)),
        _full((D_NODE, H)),
    ],
    out_specs=pl.BlockSpec((BN, H), lambda i: (i, 0)),
    out_shape=jax.ShapeDtypeStruct((N, H), jnp.float32),
)


_GATHER = _make_gather(H)
_EDGE_TC = [
    None,
    _make_edge_tc(H, H, True),
    _make_edge_tc(H, H, False),
]


def _blockdiag(w):
    z = jnp.zeros_like(w)
    return jnp.concatenate(
        [jnp.concatenate([w, z], axis=1), jnp.concatenate([z, w], axis=1)],
        axis=0)
_NODE_TC = [
    _make_node_tc(D_NODE, H, True, True),
    _make_node_tc(H, H, True, True),
    _make_node_tc(H, D_OUT, False, False),
]


def kernel(x, edge_index, edge_attr, params):
    row = edge_index[0].reshape(NW, NCHUNK, CHUNK)
    col = edge_index[1]
    col3 = col.reshape(NW, NCHUNK, CHUNK)
    col2 = col.reshape(NW, EPW)
    zeros2d = jnp.zeros((N, H), jnp.float32)
    zeros_n = jnp.zeros((N,), jnp.float32)

    (cntp,) = _counts_call(col2, zeros_n)
    cntp = cntp.T

    dxs = [D_NODE, H, H]
    xl = x
    e2 = edge_attr.reshape(EP, 2 * D_EDGE)
    btab = _btab_call(x, params[0]["We"][dxs[0]:2 * dxs[0]])
    for i, p in enumerate(params):
        dx = dxs[i]
        we, w1, w2 = p["We"], p["W1"], p["W2"]
        wsrc = _blockdiag(we[:dx])
        wee = _blockdiag(we[2 * dx:])
        w1x = _blockdiag(w1[:dx])
        w1e = _blockdiag(w1[dx:])
        w2x = w2[:dx]
        w2a = w2[dx:]
        be = jnp.tile(p["be"].reshape(1, -1), (1, 2))
        b1 = jnp.tile(p["b1"].reshape(1, -1), (1, 2))
        b2 = p["b2"].reshape(1, -1)

        if i == 0:
            gxa, gxb, gb = _gather3_call(xl[:, :H], xl[:, H:], btab, row, col3)
            enew2, m2 = _edge1_tc_call(
                gxa.reshape(EP, 2 * H), gxb.reshape(EP, 2 * H),
                gb.reshape(EP, 2 * H), e2,
                _blockdiag(we[:H]), _blockdiag(we[H:dx]), wee,
                _blockdiag(w1[:H]), _blockdiag(w1[H:dx]), w1e, be, b1)
        else:
            gx, gb = _GATHER(xl, btab, row, col3)
            gx2 = gx.reshape(EP, 2 * dx)
            gb2 = gb.reshape(EP, 2 * H)
            enew2, m2 = _EDGE_TC[i](gx2, gb2, e2, wsrc, wee, w1x, w1e, be, b1)
        (sp,) = _scatter_call(m2.reshape(E, H), col3, zeros2d)
        if i + 1 < len(params):
            wdstn = params[i + 1]["We"][dxs[i + 1]:2 * dxs[i + 1]]
        else:
            wdstn = jnp.zeros((D_OUT, H), jnp.float32)
        xl, btab = _NODE_TC[i](sp, cntp, xl, w2x, w2a, b2, wdstn)
        e2 = enew2

    return xl, e2.reshape(E, H)
